# Initial kernel scaffold; baseline (speedup 1.0000x reference)
import jax
import jax.numpy as jnp
from jax.experimental import pallas as pl


def kernel(loss, dummy):
    raise NotImplementedError("write your pallas kernel here")


# speedup vs baseline: 29.5319x; 29.5319x over previous
import jax
import jax.numpy as jnp
from jax.experimental import pallas as pl
from jax.experimental.pallas import tpu as pltpu

_B = 64
_P = 32768
_K = 8192


def _body(x_ref, out_ref):
    x = x_ref[...]
    i = jax.lax.bitcast_convert_type(x, jnp.int32)
    s = i ^ ((i >> 31) & jnp.int32(0x7FFFFFFF))

    def step(_, lohi):
        lo, hi = lohi
        mid = (lo >> 1) + (hi >> 1) + (lo & hi & 1) + ((lo ^ hi) & 1)
        cnt = jnp.sum((s >= mid).astype(jnp.int32), axis=1, keepdims=True)
        pred = cnt >= _K
        lo = jnp.where(pred, mid, lo)
        hi = jnp.where(pred, hi, mid - 1)
        return lo, hi

    lo0 = jnp.full((_B, 1), jnp.iinfo(jnp.int32).min, jnp.int32)
    hi0 = jnp.full((_B, 1), jnp.iinfo(jnp.int32).max, jnp.int32)
    t, _ = jax.lax.fori_loop(0, 32, step, (lo0, hi0))

    gt = s > t
    cnt_gt = jnp.sum(gt.astype(jnp.float32), axis=1, keepdims=True)
    sum_gt = jnp.sum(jnp.where(gt, x, 0.0), axis=1, keepdims=True)
    tf = jax.lax.bitcast_convert_type(t ^ ((t >> 31) & jnp.int32(0x7FFFFFFF)),
                                      jnp.float32)
    row = sum_gt + (jnp.float32(_K) - cnt_gt) * tf
    out_ref[0, 0] = jnp.sum(row) / jnp.float32(_B * _K)


def kernel(loss, dummy):
    out = pl.pallas_call(
        _body,
        out_shape=jax.ShapeDtypeStruct((1, 1), jnp.float32),
        out_specs=pl.BlockSpec(memory_space=pltpu.SMEM),
    )(loss)
    return out[0, 0]
